# baseline (speedup 1.0000x reference)
import functools

import jax
import jax.numpy as jnp
from jax import lax
from jax.experimental import pallas as pl
from jax.experimental.pallas import tpu as pltpu
from jax.experimental.pallas import tpu_sc as plsc

_NC = 2
_NS = 16
_NW = _NC * _NS
_LANES = 16


def _noisy_body(emb_ref, w_ref, b_ref, eps_ref, noisy_ref):
    E = eps_ref.shape[0]
    rl = lax.dot_general(
        w_ref[...], emb_ref[...],
        dimension_numbers=(((1,), (1,)), ((), ())),
        preferred_element_type=jnp.float32,
    ) + b_ref[...]
    route = rl[:E, :]
    noise = rl[E:, :]
    noisy_ref[...] = route + eps_ref[...] * jax.nn.softplus(noise)


def _sc_route_body(E, tpw, noisy_hbm, p_hbm, rank_hbm, maxes_hbm,
                   nbuf, pbuf, rbuf, mbuf):
    wid = lax.axis_index("s") * _NC + lax.axis_index("c")
    base = wid * tpw
    pltpu.sync_copy(noisy_hbm.at[:, pl.ds(base, tpw)], nbuf)
    tnmax = jnp.zeros((_LANES,), jnp.int32)
    one = jnp.ones((_LANES,), jnp.int32)
    izero = jnp.zeros((_LANES,), jnp.int32)
    zero = jnp.zeros((_LANES,), jnp.float32)
    for g in range(tpw // _LANES):
        sl = pl.ds(g * _LANES, _LANES)
        vecs = [nbuf[e, sl] for e in range(E)]
        mx = vecs[0]
        for e in range(1, E):
            mx = jnp.maximum(mx, vecs[e])
        exps = [jnp.exp(v - mx) for v in vecs]
        ssum = exps[0]
        for e in range(1, E):
            ssum = ssum + exps[e]
        ps = [ex / ssum for ex in exps]
        ranks = [izero for _ in range(E)]
        cums = list(ps)
        for i in range(E):
            for j in range(i + 1, E):
                b = ps[i] >= ps[j]
                ranks[j] = ranks[j] + jnp.where(b, one, izero)
                ranks[i] = ranks[i] + jnp.where(b, izero, one)
                cums[j] = cums[j] + jnp.where(b, ps[i], zero)
                cums[i] = cums[i] + jnp.where(b, zero, ps[j])
        tn = izero
        for j in range(E):
            m = (cums[j] < 0.5) | (ranks[j] == 0)
            tn = tn + jnp.where(m, one, izero)
        tnmax = jnp.maximum(tnmax, tn)
        for e in range(E):
            pbuf[e, sl] = ps[e]
            rbuf[e, sl] = ranks[e]
    mbuf[...] = tnmax
    pltpu.sync_copy(pbuf, p_hbm.at[:, pl.ds(base, tpw)])
    pltpu.sync_copy(rbuf, rank_hbm.at[:, pl.ds(base, tpw)])
    pltpu.sync_copy(mbuf, maxes_hbm.at[pl.ds(wid * _LANES, _LANES)])


def _combine_body(maxes_ref, logits_ref, p_ref, rank_ref, out_ref):
    E = logits_ref.shape[0]
    max_k = jnp.max(maxes_ref[...])
    p = p_ref[...]
    sel = (rank_ref[...] < max_k).astype(jnp.float32)
    tw = p * sel
    u = tw / (jnp.sum(tw, axis=0, keepdims=True) + 1e-6)
    acc = logits_ref[0] * u[0, :, None]
    for e in range(1, E):
        acc = acc + logits_ref[e] * u[e, :, None]
    out_ref[...] = acc


def kernel(embedding, logits, W_route, b_route, W_noise, b_noise):
    B, S, H = embedding.shape
    E, V = logits.shape[0], logits.shape[-1]
    T = B * S
    emb = embedding.reshape(T, H)
    w_cat = jnp.concatenate([W_route, W_noise], axis=0)
    b_cat = jnp.concatenate([b_route, b_noise]).reshape(2 * E, 1)
    eps_t = jax.random.normal(
        jax.random.fold_in(jax.random.key(0), 123), (B, S, E), jnp.float32
    ).reshape(T, E).T

    Tr = 512
    while T % Tr:
        Tr //= 2
    noisy_t = pl.pallas_call(
        _noisy_body,
        grid=(T // Tr,),
        in_specs=[
            pl.BlockSpec((Tr, H), lambda i: (i, 0)),
            pl.BlockSpec((2 * E, H), lambda i: (0, 0)),
            pl.BlockSpec((2 * E, 1), lambda i: (0, 0)),
            pl.BlockSpec((E, Tr), lambda i: (0, i)),
        ],
        out_specs=pl.BlockSpec((E, Tr), lambda i: (0, i)),
        out_shape=jax.ShapeDtypeStruct((E, T), jnp.float32),
        compiler_params=pltpu.CompilerParams(
            dimension_semantics=("arbitrary",),
        ),
    )(emb, w_cat, b_cat, eps_t)

    tpw = T // _NW
    mesh = plsc.VectorSubcoreMesh(
        core_axis_name="c", subcore_axis_name="s",
        num_cores=_NC, num_subcores=_NS,
    )
    sc_route = functools.partial(
        pl.kernel,
        out_type=[
            jax.ShapeDtypeStruct((E, T), jnp.float32),
            jax.ShapeDtypeStruct((E, T), jnp.int32),
            jax.ShapeDtypeStruct((_NW * _LANES,), jnp.int32),
        ],
        mesh=mesh,
        scratch_types=[
            pltpu.VMEM((E, tpw), jnp.float32),
            pltpu.VMEM((E, tpw), jnp.float32),
            pltpu.VMEM((E, tpw), jnp.int32),
            pltpu.VMEM((_LANES,), jnp.int32),
        ],
    )(functools.partial(_sc_route_body, E, tpw))
    p_t, rank_t, maxes = sc_route(noisy_t)

    maxes2d = maxes.reshape(_NW, _LANES)

    Ts = 512
    while T % Ts:
        Ts //= 2
    lg = logits.reshape(E, T, V)
    combined = pl.pallas_call(
        _combine_body,
        grid=(T // Ts,),
        in_specs=[
            pl.BlockSpec((_NW, _LANES), lambda i: (0, 0)),
            pl.BlockSpec((E, Ts, V), lambda i: (0, i, 0)),
            pl.BlockSpec((E, Ts), lambda i: (0, i)),
            pl.BlockSpec((E, Ts), lambda i: (0, i)),
        ],
        out_specs=pl.BlockSpec((Ts, V), lambda i: (i, 0)),
        out_shape=jax.ShapeDtypeStruct((T, V), jnp.float32),
        compiler_params=pltpu.CompilerParams(
            dimension_semantics=("arbitrary",),
        ),
    )(maxes2d, lg, p_t, rank_t)

    route_prob = p_t.T.reshape(B, S, E)
    return combined.reshape(B, S, V), route_prob, route_prob


# speedup vs baseline: 1.0249x; 1.0249x over previous
import functools

import jax
import jax.numpy as jnp
from jax import lax
from jax.experimental import pallas as pl
from jax.experimental.pallas import tpu as pltpu
from jax.experimental.pallas import tpu_sc as plsc

_NC = 2
_NS = 16
_NW = _NC * _NS
_LANES = 16


def _noisy_body(emb_ref, w_ref, b_ref, eps_ref, noisy_ref):
    E = eps_ref.shape[0]
    rl = lax.dot_general(
        w_ref[...], emb_ref[...],
        dimension_numbers=(((1,), (1,)), ((), ())),
        preferred_element_type=jnp.float32,
    ) + b_ref[...]
    route = rl[:E, :]
    noise = rl[E:, :]
    noisy_ref[...] = route + eps_ref[...] * jax.nn.softplus(noise)


def _sc_route_body(E, tpw, noisy_hbm, p_hbm, rank_hbm, maxes_hbm,
                   nbuf, pbuf, rbuf, mbuf):
    wid = lax.axis_index("s") * _NC + lax.axis_index("c")
    base = wid * tpw
    pltpu.sync_copy(noisy_hbm.at[:, pl.ds(base, tpw)], nbuf)
    tnmax = jnp.zeros((_LANES,), jnp.int32)
    one = jnp.ones((_LANES,), jnp.int32)
    izero = jnp.zeros((_LANES,), jnp.int32)
    zero = jnp.zeros((_LANES,), jnp.float32)
    for g in range(tpw // _LANES):
        sl = pl.ds(g * _LANES, _LANES)
        vecs = [nbuf[e, sl] for e in range(E)]
        mx = vecs[0]
        for e in range(1, E):
            mx = jnp.maximum(mx, vecs[e])
        exps = [jnp.exp(v - mx) for v in vecs]
        ssum = exps[0]
        for e in range(1, E):
            ssum = ssum + exps[e]
        ps = [ex / ssum for ex in exps]
        ranks = [izero for _ in range(E)]
        cums = list(ps)
        for i in range(E):
            for j in range(i + 1, E):
                b = ps[i] >= ps[j]
                ranks[j] = ranks[j] + jnp.where(b, one, izero)
                ranks[i] = ranks[i] + jnp.where(b, izero, one)
                cums[j] = cums[j] + jnp.where(b, ps[i], zero)
                cums[i] = cums[i] + jnp.where(b, zero, ps[j])
        tn = izero
        for j in range(E):
            m = (cums[j] < 0.5) | (ranks[j] == 0)
            tn = tn + jnp.where(m, one, izero)
        tnmax = jnp.maximum(tnmax, tn)
        for e in range(E):
            pbuf[e, sl] = ps[e]
            rbuf[e, sl] = ranks[e]
    mbuf[...] = tnmax
    pltpu.sync_copy(pbuf, p_hbm.at[:, pl.ds(base, tpw)])
    pltpu.sync_copy(rbuf, rank_hbm.at[:, pl.ds(base, tpw)])
    pltpu.sync_copy(mbuf, maxes_hbm.at[pl.ds(wid * _LANES, _LANES)])


def _combine_body(maxes_ref, logits_ref, p_ref, rank_ref, out_ref):
    E = logits_ref.shape[0]
    max_k = jnp.max(maxes_ref[...])
    p = p_ref[...]
    sel = (rank_ref[...] < max_k).astype(jnp.float32)
    tw = p * sel
    u = tw / (jnp.sum(tw, axis=0, keepdims=True) + 1e-6)
    acc = logits_ref[0] * u[0, :, None]
    for e in range(1, E):
        acc = acc + logits_ref[e] * u[e, :, None]
    out_ref[...] = acc


def kernel(embedding, logits, W_route, b_route, W_noise, b_noise):
    B, S, H = embedding.shape
    E, V = logits.shape[0], logits.shape[-1]
    T = B * S
    emb = embedding.reshape(T, H)
    w_cat = jnp.concatenate([W_route, W_noise], axis=0)
    b_cat = jnp.concatenate([b_route, b_noise]).reshape(2 * E, 1)
    eps_t = jax.random.normal(
        jax.random.fold_in(jax.random.key(0), 123), (B, S, E), jnp.float32
    ).reshape(T, E).T

    Tr = 1024
    while T % Tr:
        Tr //= 2
    noisy_t = pl.pallas_call(
        _noisy_body,
        grid=(T // Tr,),
        in_specs=[
            pl.BlockSpec((Tr, H), lambda i: (i, 0)),
            pl.BlockSpec((2 * E, H), lambda i: (0, 0)),
            pl.BlockSpec((2 * E, 1), lambda i: (0, 0)),
            pl.BlockSpec((E, Tr), lambda i: (0, i)),
        ],
        out_specs=pl.BlockSpec((E, Tr), lambda i: (0, i)),
        out_shape=jax.ShapeDtypeStruct((E, T), jnp.float32),
        compiler_params=pltpu.CompilerParams(
            dimension_semantics=("arbitrary",),
        ),
    )(emb, w_cat, b_cat, eps_t)

    tpw = T // _NW
    mesh = plsc.VectorSubcoreMesh(
        core_axis_name="c", subcore_axis_name="s",
        num_cores=_NC, num_subcores=_NS,
    )
    sc_route = functools.partial(
        pl.kernel,
        out_type=[
            jax.ShapeDtypeStruct((E, T), jnp.float32),
            jax.ShapeDtypeStruct((E, T), jnp.int32),
            jax.ShapeDtypeStruct((_NW * _LANES,), jnp.int32),
        ],
        mesh=mesh,
        scratch_types=[
            pltpu.VMEM((E, tpw), jnp.float32),
            pltpu.VMEM((E, tpw), jnp.float32),
            pltpu.VMEM((E, tpw), jnp.int32),
            pltpu.VMEM((_LANES,), jnp.int32),
        ],
    )(functools.partial(_sc_route_body, E, tpw))
    p_t, rank_t, maxes = sc_route(noisy_t)

    maxes2d = maxes.reshape(_NW, _LANES)

    Ts = 512
    while T % Ts:
        Ts //= 2
    lg = logits.reshape(E, T, V)
    combined = pl.pallas_call(
        _combine_body,
        grid=(T // Ts,),
        in_specs=[
            pl.BlockSpec((_NW, _LANES), lambda i: (0, 0)),
            pl.BlockSpec((E, Ts, V), lambda i: (0, i, 0)),
            pl.BlockSpec((E, Ts), lambda i: (0, i)),
            pl.BlockSpec((E, Ts), lambda i: (0, i)),
        ],
        out_specs=pl.BlockSpec((Ts, V), lambda i: (i, 0)),
        out_shape=jax.ShapeDtypeStruct((T, V), jnp.float32),
        compiler_params=pltpu.CompilerParams(
            dimension_semantics=("arbitrary",),
        ),
    )(maxes2d, lg, p_t, rank_t)

    route_prob = p_t.T.reshape(B, S, E)
    return combined.reshape(B, S, V), route_prob, route_prob
